# baseline (speedup 1.0000x reference)
import functools

import jax
import jax.numpy as jnp
from jax import lax
from jax.experimental import pallas as pl
from jax.experimental.pallas import tpu as pltpu
from jax.experimental.pallas import tpu_sc as plsc

VOCAB = 1000000
EMBED_DIM = 64
PAD_LEN = 200
BATCH = 4096
CLASS_NUM = 16

_D = EMBED_DIM
_L = PAD_LEN
_NC = 2
_NS = 16
_NW = _NC * _NS
_BW = BATCH // _NW
_CW = 40
_CPR = _L // _CW
_NCH = _BW * _CPR
_NB = 16

_CB = 32768
_NBLK = (VOCAB + _CB - 1) // _CB
_VPAD = _NBLK * _CB


def _fire(table_hbm, tex_v, buf, sem, c):
    r = c // _CPR
    off = pl.multiple_of(_CW * (c - _CPR * r), 8)
    pltpu.make_async_copy(table_hbm.at[tex_v.at[r, pl.ds(off, _CW)]],
                          buf, sem).start()


def _drain(table_hbm, buf, sem):
    pltpu.make_async_copy(table_hbm.at[pl.ds(0, _CW)], buf, sem).wait()


def _accum(buf, acc_v, row):
    z = jnp.zeros((16,), jnp.float32)

    def body(t, accs):
        a0, a1, a2, a3 = accs
        for u in range(4):
            j = 4 * t + u
            a0 = a0 + buf[j, pl.ds(0, 16)]
            a1 = a1 + buf[j, pl.ds(16, 16)]
            a2 = a2 + buf[j, pl.ds(32, 16)]
            a3 = a3 + buf[j, pl.ds(48, 16)]
        return (a0, a1, a2, a3)

    a0, a1, a2, a3 = lax.fori_loop(0, _CW // 4, body, (z, z, z, z))
    plsc.addupdate(acc_v.at[row, pl.ds(0, 16)], a0)
    plsc.addupdate(acc_v.at[row, pl.ds(16, 16)], a1)
    plsc.addupdate(acc_v.at[row, pl.ds(32, 16)], a2)
    plsc.addupdate(acc_v.at[row, pl.ds(48, 16)], a3)


@functools.partial(
    pl.kernel,
    mesh=plsc.VectorSubcoreMesh(core_axis_name="c", subcore_axis_name="s"),
    out_type=jax.ShapeDtypeStruct((BATCH, _D), jnp.float32),
    compiler_params=pltpu.CompilerParams(use_tc_tiling_on_sc=False),
    scratch_types=(
        [pltpu.VMEM((_BW, _L), jnp.int32)]
        + [pltpu.VMEM((_BW, _D), jnp.float32)]
        + [pltpu.VMEM((_CW, _D), jnp.float32)] * _NB
        + [pltpu.SemaphoreType.DMA] * _NB
    ),
)
def _sc_lookup_sum(texts_hbm, table_hbm, out_hbm, tex_v, acc_v, *ring):
    bufs, sems = ring[:_NB], ring[_NB:]
    wid = lax.axis_index("s") * _NC + lax.axis_index("c")
    base = wid * _BW
    pltpu.sync_copy(texts_hbm.at[pl.ds(base, _BW)], tex_v)

    z = jnp.zeros((16,), jnp.float32)

    def zero_body(t, carry):
        for u in range(16):
            acc_v[4 * t + u // 4, pl.ds(16 * (u % 4), 16)] = z
        return carry
    lax.fori_loop(0, (_BW * _D) // 256, zero_body, 0)

    for b in range(_NB):
        _fire(table_hbm, tex_v, bufs[b], sems[b], b)

    def outer(g, carry):
        for b in range(_NB):
            c = g * _NB + b
            _drain(table_hbm, bufs[b], sems[b])
            _accum(bufs[b], acc_v, c // _CPR)

            @pl.when(c + _NB < _NCH)
            def _():
                _fire(table_hbm, tex_v, bufs[b], sems[b], c + _NB)
        return carry

    lax.fori_loop(0, _NCH // _NB, outer, 0)
    pltpu.sync_copy(acc_v, out_hbm.at[pl.ds(base, _BW)])


def _conv_body(x_ref, o_ref):
    eye = jnp.float32(
        lax.broadcasted_iota(jnp.int32, (_D, _D), 0)
        == lax.broadcasted_iota(jnp.int32, (_D, _D), 1))
    y = lax.dot_general(x_ref[...], eye, (((0,), (0,)), ((), ())),
                        preferred_element_type=jnp.float32)
    o_ref[:, 0:64] = y[0:_CB // 2, :]
    o_ref[:, 64:128] = y[_CB // 2:_CB, :]


_conv_call = pl.pallas_call(
    _conv_body,
    grid=(_NBLK,),
    in_specs=[pl.BlockSpec((_D, _CB), lambda i: (0, i))],
    out_specs=pl.BlockSpec((_CB // 2, 128), lambda i: (i, 0)),
    out_shape=jax.ShapeDtypeStruct((_VPAD // 2, 128), jnp.float32),
)


def _remap_body(x_ref, o_ref):
    v = x_ref[...]
    o_ref[...] = (
        (v & jnp.int32(-_CB))
        | ((v & jnp.int32(_CB // 2 - 1)) << 1)
        | ((v >> jnp.int32(_CB.bit_length() - 2)) & jnp.int32(1))
    )


_remap_call = pl.pallas_call(
    _remap_body,
    out_shape=jax.ShapeDtypeStruct((BATCH, _L), jnp.int32),
)


def _fc_body(x_ref, w_ref, b_ref, o_ref):
    o_ref[...] = (
        jnp.dot(x_ref[...], w_ref[...], preferred_element_type=jnp.float32)
        + b_ref[...]
    )


_fc_call = pl.pallas_call(
    _fc_body,
    out_shape=jax.ShapeDtypeStruct((BATCH, 128), jnp.float32),
)


def kernel(texts, table, fc_w, fc_b):
    table_lin = _conv_call(jnp.transpose(table)).reshape(_VPAD, _D)
    texts_m = _remap_call(texts.astype(jnp.int32))
    sums = _sc_lookup_sum(texts_m, table_lin)
    w_t = jnp.transpose(fc_w) * jnp.float32(1.0 / _L)
    w_pad = jnp.pad(w_t, ((0, 0), (0, 128 - CLASS_NUM)))
    b_pad = jnp.pad(fc_b, (0, 128 - CLASS_NUM)).reshape(1, 128)
    out = _fc_call(sums, w_pad, b_pad)
    return out[:, :CLASS_NUM]


# speedup vs baseline: 1.0413x; 1.0413x over previous
import functools

import jax
import jax.numpy as jnp
from jax import lax
from jax.experimental import pallas as pl
from jax.experimental.pallas import tpu as pltpu
from jax.experimental.pallas import tpu_sc as plsc

VOCAB = 1000000
EMBED_DIM = 64
PAD_LEN = 200
BATCH = 4096
CLASS_NUM = 16

_D = EMBED_DIM
_L = PAD_LEN
_NC = 2
_NS = 16
_NW = _NC * _NS
_BW = BATCH // _NW
_CW = 40
_CPR = _L // _CW
_NCH = _BW * _CPR
_NB = 10

_CB = 32768
_NBLK = (VOCAB + _CB - 1) // _CB
_VPAD = _NBLK * _CB


def _fire(table_hbm, tex_v, buf, sem, c):
    r = c // _CPR
    off = pl.multiple_of(_CW * (c - _CPR * r), 8)
    pltpu.make_async_copy(table_hbm.at[tex_v.at[r, pl.ds(off, _CW)]],
                          buf, sem).start()


def _drain(table_hbm, buf, sem):
    pltpu.make_async_copy(table_hbm.at[pl.ds(0, _CW)], buf, sem).wait()


def _accum(buf, acc_v, row):
    z = jnp.zeros((16,), jnp.float32)

    def body(t, accs):
        a0, a1, a2, a3 = accs
        for u in range(4):
            j = 4 * t + u
            a0 = a0 + buf[j, pl.ds(0, 16)]
            a1 = a1 + buf[j, pl.ds(16, 16)]
            a2 = a2 + buf[j, pl.ds(32, 16)]
            a3 = a3 + buf[j, pl.ds(48, 16)]
        return (a0, a1, a2, a3)

    a0, a1, a2, a3 = lax.fori_loop(0, _CW // 4, body, (z, z, z, z))
    plsc.addupdate(acc_v.at[row, pl.ds(0, 16)], a0)
    plsc.addupdate(acc_v.at[row, pl.ds(16, 16)], a1)
    plsc.addupdate(acc_v.at[row, pl.ds(32, 16)], a2)
    plsc.addupdate(acc_v.at[row, pl.ds(48, 16)], a3)


@functools.partial(
    pl.kernel,
    mesh=plsc.VectorSubcoreMesh(core_axis_name="c", subcore_axis_name="s"),
    out_type=jax.ShapeDtypeStruct((BATCH, _D), jnp.float32),
    compiler_params=pltpu.CompilerParams(use_tc_tiling_on_sc=False),
    scratch_types=(
        [pltpu.VMEM((_BW, _L), jnp.int32)]
        + [pltpu.VMEM((_BW, _D), jnp.float32)]
        + [pltpu.VMEM((_CW, _D), jnp.float32)] * _NB
        + [pltpu.SemaphoreType.DMA] * _NB
    ),
)
def _sc_lookup_sum(texts_hbm, table_hbm, out_hbm, tex_v, acc_v, *ring):
    bufs, sems = ring[:_NB], ring[_NB:]
    wid = lax.axis_index("s") * _NC + lax.axis_index("c")
    base = wid * _BW
    pltpu.sync_copy(texts_hbm.at[pl.ds(base, _BW)], tex_v)

    z = jnp.zeros((16,), jnp.float32)

    def zero_body(t, carry):
        for u in range(16):
            acc_v[4 * t + u // 4, pl.ds(16 * (u % 4), 16)] = z
        return carry
    lax.fori_loop(0, (_BW * _D) // 256, zero_body, 0)

    for b in range(_NB):
        _fire(table_hbm, tex_v, bufs[b], sems[b], b)

    def outer(g, carry):
        for b in range(_NB):
            c = g * _NB + b
            _drain(table_hbm, bufs[b], sems[b])
            _accum(bufs[b], acc_v, c // _CPR)

            @pl.when(c + _NB < _NCH)
            def _():
                _fire(table_hbm, tex_v, bufs[b], sems[b], c + _NB)
        return carry

    lax.fori_loop(0, _NCH // _NB, outer, 0)
    pltpu.sync_copy(acc_v, out_hbm.at[pl.ds(base, _BW)])


def _conv_body(x_ref, o_ref):
    eye = jnp.float32(
        lax.broadcasted_iota(jnp.int32, (_D, _D), 0)
        == lax.broadcasted_iota(jnp.int32, (_D, _D), 1))
    y = lax.dot_general(x_ref[...], eye, (((0,), (0,)), ((), ())),
                        preferred_element_type=jnp.float32)
    o_ref[:, 0:64] = y[0:_CB // 2, :]
    o_ref[:, 64:128] = y[_CB // 2:_CB, :]


_conv_call = pl.pallas_call(
    _conv_body,
    grid=(_NBLK,),
    in_specs=[pl.BlockSpec((_D, _CB), lambda i: (0, i))],
    out_specs=pl.BlockSpec((_CB // 2, 128), lambda i: (i, 0)),
    out_shape=jax.ShapeDtypeStruct((_VPAD // 2, 128), jnp.float32),
)


def _remap_body(x_ref, o_ref):
    v = x_ref[...]
    o_ref[...] = (
        (v & jnp.int32(-_CB))
        | ((v & jnp.int32(_CB // 2 - 1)) << 1)
        | ((v >> jnp.int32(_CB.bit_length() - 2)) & jnp.int32(1))
    )


_remap_call = pl.pallas_call(
    _remap_body,
    out_shape=jax.ShapeDtypeStruct((BATCH, _L), jnp.int32),
)


def _fc_body(x_ref, w_ref, b_ref, o_ref):
    o_ref[...] = (
        jnp.dot(x_ref[...], w_ref[...], preferred_element_type=jnp.float32)
        + b_ref[...]
    )


_fc_call = pl.pallas_call(
    _fc_body,
    out_shape=jax.ShapeDtypeStruct((BATCH, 128), jnp.float32),
)


def kernel(texts, table, fc_w, fc_b):
    table_lin = _conv_call(jnp.transpose(table)).reshape(_VPAD, _D)
    texts_m = _remap_call(texts.astype(jnp.int32))
    sums = _sc_lookup_sum(texts_m, table_lin)
    w_t = jnp.transpose(fc_w) * jnp.float32(1.0 / _L)
    w_pad = jnp.pad(w_t, ((0, 0), (0, 128 - CLASS_NUM)))
    b_pad = jnp.pad(fc_b, (0, 128 - CLASS_NUM)).reshape(1, 128)
    out = _fc_call(sums, w_pad, b_pad)
    return out[:, :CLASS_NUM]
